# baseline (speedup 1.0000x reference)
import jax
import jax.numpy as jnp
from jax.experimental import pallas as pl
from jax.experimental.pallas import tpu as pltpu


CHUNK = 1024
NBUF = 6


def _mlp_kernel(x_hbm, w_ref, b_ref, o_hbm, xbuf, obuf, load_sem, store_sem):
    n = x_hbm.shape[0]
    nchunks = n // CHUNK

    def load(i, slot):
        return pltpu.make_async_copy(
            x_hbm.at[pl.ds(i * CHUNK, CHUNK), :], xbuf.at[slot],
            load_sem.at[slot])

    def store(i, slot):
        return pltpu.make_async_copy(
            obuf.at[slot], o_hbm.at[pl.ds(i * CHUNK, CHUNK), :],
            store_sem.at[slot])

    for k in range(min(NBUF, nchunks)):
        load(k, k).start()

    for i in range(nchunks):
        slot = i % NBUF
        load(i, slot).wait()
        if i >= NBUF:
            store(i - NBUF, slot).wait()
        acc = jax.lax.dot_general(
            xbuf[slot], w_ref[...],
            dimension_numbers=(((1,), (1,)), ((), ())),
            preferred_element_type=jnp.float32,
        )
        obuf[slot] = acc + b_ref[...]
        store(i, slot).start()
        if i + NBUF < nchunks:
            load(i + NBUF, slot).start()

    for i in range(max(nchunks - NBUF, 0), nchunks):
        store(i, i % NBUF).wait()


@jax.jit
def kernel(x, W, b):
    n, hidden = x.shape
    out_dim = W.shape[0]
    b2 = b.reshape(1, out_dim)
    return pl.pallas_call(
        _mlp_kernel,
        in_specs=[
            pl.BlockSpec(memory_space=pl.ANY),
            pl.BlockSpec(memory_space=pltpu.VMEM),
            pl.BlockSpec(memory_space=pltpu.VMEM),
        ],
        out_specs=pl.BlockSpec(memory_space=pl.ANY),
        out_shape=jax.ShapeDtypeStruct((n, out_dim), jnp.float32),
        scratch_shapes=[
            pltpu.VMEM((NBUF, CHUNK, out_dim), jnp.float32),
            pltpu.VMEM((NBUF, CHUNK, out_dim), jnp.float32),
            pltpu.SemaphoreType.DMA((NBUF,)),
            pltpu.SemaphoreType.DMA((NBUF,)),
        ],
    )(x, W, b2)


# speedup vs baseline: 1.1108x; 1.1108x over previous
import jax
import jax.numpy as jnp
from jax.experimental import pallas as pl
from jax.experimental.pallas import tpu as pltpu


CHUNK = 1024
NBUF = 3


def _mlp_kernel(x_hbm, w_ref, b_ref, o_hbm, xbuf, obuf, load_sem, store_sem):
    n = x_hbm.shape[0]
    nchunks = n // CHUNK

    def load(i, slot):
        return pltpu.make_async_copy(
            x_hbm.at[pl.ds(i * CHUNK, CHUNK), :], xbuf.at[slot],
            load_sem.at[slot])

    def store(i, slot):
        return pltpu.make_async_copy(
            obuf.at[slot], o_hbm.at[pl.ds(i * CHUNK, CHUNK), :],
            store_sem.at[slot])

    for k in range(min(NBUF, nchunks)):
        load(k, k).start()

    for i in range(nchunks):
        slot = i % NBUF
        load(i, slot).wait()
        if i >= NBUF:
            store(i - NBUF, slot).wait()
        acc = jax.lax.dot_general(
            xbuf[slot], w_ref[...],
            dimension_numbers=(((1,), (1,)), ((), ())),
            preferred_element_type=jnp.float32,
        )
        obuf[slot] = acc + b_ref[...]
        store(i, slot).start()
        if i + NBUF < nchunks:
            load(i + NBUF, slot).start()

    for i in range(max(nchunks - NBUF, 0), nchunks):
        store(i, i % NBUF).wait()


@jax.jit
def kernel(x, W, b):
    n, hidden = x.shape
    out_dim = W.shape[0]
    b2 = b.reshape(1, out_dim)
    return pl.pallas_call(
        _mlp_kernel,
        in_specs=[
            pl.BlockSpec(memory_space=pl.ANY),
            pl.BlockSpec(memory_space=pltpu.VMEM),
            pl.BlockSpec(memory_space=pltpu.VMEM),
        ],
        out_specs=pl.BlockSpec(memory_space=pl.ANY),
        out_shape=jax.ShapeDtypeStruct((n, out_dim), jnp.float32),
        scratch_shapes=[
            pltpu.VMEM((NBUF, CHUNK, out_dim), jnp.float32),
            pltpu.VMEM((NBUF, CHUNK, out_dim), jnp.float32),
            pltpu.SemaphoreType.DMA((NBUF,)),
            pltpu.SemaphoreType.DMA((NBUF,)),
        ],
    )(x, W, b2)
